# baseline (speedup 1.0000x reference)
import jax
import jax.numpy as jnp
from jax import lax
from jax.experimental import pallas as pl
from jax.experimental.pallas import tpu as pltpu
from jax.experimental.pallas import tpu_sc as plsc

NC = 2
NS = 16
NW = NC * NS
L = 16

D = 768
SEQ = 8192
BATCH = 4
ROWS_PER_W = SEQ // NW
CH = 32
GR = 8
NB = 3
NCHUNK = ROWS_PER_W // CH
NSTEP = NCHUNK * BATCH


def _body(x_hbm, abs_hbm, m0_hbm, m1_hbm, m2_hbm, out_hbm,
          m0_v, m1_v, m2_v, c12_v, pe_v, xb0_v, xb1_v, xb2_v,
          sem_abs, sem_x0, sem_x1, sem_x2, sem_o0, sem_o1, sem_o2):
    wid = lax.axis_index("s") * NC + lax.axis_index("c")
    base = wid * ROWS_PER_W

    xb = (xb0_v, xb1_v, xb2_v)
    sem_x = (sem_x0, sem_x1, sem_x2)
    sem_o = (sem_o0, sem_o1, sem_o2)

    pltpu.sync_copy(m0_hbm, m0_v)
    pltpu.sync_copy(m1_hbm, m1_v)
    pltpu.sync_copy(m2_hbm, m2_v)
    for r in range(12):
        @plsc.parallel_loop(0, D, step=L, unroll=8)
        def c12_vec(off, r=r):
            c12_v[r, pl.ds(off, L)] = (
                m0_v[r % 2, pl.ds(off, L)]
                + m1_v[r % 3, pl.ds(off, L)]
                + m2_v[r % 4, pl.ds(off, L)]
            )

    def abs_load(c):
        row0 = base + c * CH
        return pltpu.async_copy(abs_hbm.at[pl.ds(row0, CH)], pe_v, sem_abs)

    def x_load(gb):
        c, b = divmod(gb, BATCH)
        row0 = base + c * CH
        return pltpu.async_copy(x_hbm.at[b, pl.ds(row0, CH)], xb[gb % NB],
                                sem_x[gb % NB])

    def out_store(gb):
        c, b = divmod(gb, BATCH)
        row0 = base + c * CH
        return pltpu.async_copy(xb[gb % NB], out_hbm.at[b, pl.ds(row0, CH)],
                                sem_o[gb % NB])

    d_abs = abs_load(0)
    d_x = [None] * NB
    d_o = [None] * NB
    for g in range(NB - 1):
        d_x[g] = x_load(g)

    for c in range(NCHUNK):
        d_abs.wait()

        p0 = (base + c * CH) % 12

        @plsc.parallel_loop(0, CH // GR, step=1)
        def pe_grp(g):
            js = [(p0 + g * GR + k) % 12 for k in range(GR)]

            @plsc.parallel_loop(0, D, step=L, unroll=2)
            def pe_vec(off):
                for k in range(GR):
                    plsc.addupdate(pe_v.at[g * GR + k, pl.ds(off, L)],
                                   c12_v[js[k], pl.ds(off, L)])

        for b in range(BATCH):
            gb = c * BATCH + b
            A = gb % NB
            d_x[A].wait()
            if gb + NB - 1 < NSTEP:
                B = (gb + NB - 1) % NB
                if d_o[B] is not None:
                    d_o[B].wait()
                    d_o[B] = None
                d_x[B] = x_load(gb + NB - 1)

            @plsc.parallel_loop(0, CH // GR, step=1)
            def add_grp(g):
                @plsc.parallel_loop(0, D, step=L, unroll=2)
                def add_vec(off):
                    for k in range(GR):
                        plsc.addupdate(xb[A].at[g * GR + k, pl.ds(off, L)],
                                       pe_v[g * GR + k, pl.ds(off, L)])

            if b == BATCH - 1 and c + 1 < NCHUNK:
                d_abs = abs_load(c + 1)

            d_o[A] = out_store(gb)

    for i in range(NB):
        if d_o[i] is not None:
            d_o[i].wait()


@jax.jit
def kernel(x, abs_table, mod_table_0, mod_table_1, mod_table_2):
    mesh = plsc.VectorSubcoreMesh(core_axis_name="c", subcore_axis_name="s")
    f = pl.kernel(
        _body,
        out_type=jax.ShapeDtypeStruct((BATCH, SEQ, D), jnp.float32),
        mesh=mesh,
        scratch_types=[
            pltpu.VMEM((2, D), jnp.float32),
            pltpu.VMEM((3, D), jnp.float32),
            pltpu.VMEM((4, D), jnp.float32),
            pltpu.VMEM((12, D), jnp.float32),
            pltpu.VMEM((CH, D), jnp.float32),
            pltpu.VMEM((CH, D), jnp.float32),
            pltpu.VMEM((CH, D), jnp.float32),
            pltpu.VMEM((CH, D), jnp.float32),
            pltpu.SemaphoreType.DMA,
            pltpu.SemaphoreType.DMA,
            pltpu.SemaphoreType.DMA,
            pltpu.SemaphoreType.DMA,
            pltpu.SemaphoreType.DMA,
            pltpu.SemaphoreType.DMA,
            pltpu.SemaphoreType.DMA,
        ],
    )
    return f(x, abs_table, mod_table_0, mod_table_1, mod_table_2)


# speedup vs baseline: 1.0800x; 1.0800x over previous
import jax
import jax.numpy as jnp
from jax import lax
from jax.experimental import pallas as pl
from jax.experimental.pallas import tpu as pltpu
from jax.experimental.pallas import tpu_sc as plsc

NC = 2
NS = 16
NW = NC * NS
L = 16

D = 768
SEQ = 8192
BATCH = 4
ROWS_PER_W = SEQ // NW
CH = 32
GR = 8
NCHUNK = ROWS_PER_W // CH
NSTEP = NCHUNK * BATCH


def _body(x_hbm, abs_hbm, m0_hbm, m1_hbm, m2_hbm, out_hbm,
          m0_v, m1_v, m2_v, c12_v, pe0_v, pe1_v, xb0_v, xb1_v,
          sem_abs0, sem_abs1, sem_x0, sem_x1, sem_o0, sem_o1):
    wid = lax.axis_index("s") * NC + lax.axis_index("c")
    base = wid * ROWS_PER_W

    pe = (pe0_v, pe1_v)
    xb = (xb0_v, xb1_v)
    sem_abs = (sem_abs0, sem_abs1)
    sem_x = (sem_x0, sem_x1)
    sem_o = (sem_o0, sem_o1)

    pltpu.sync_copy(m0_hbm, m0_v)
    pltpu.sync_copy(m1_hbm, m1_v)
    pltpu.sync_copy(m2_hbm, m2_v)
    for r in range(12):
        @plsc.parallel_loop(0, D, step=L, unroll=8)
        def c12_vec(off, r=r):
            c12_v[r, pl.ds(off, L)] = (
                m0_v[r % 2, pl.ds(off, L)]
                + m1_v[r % 3, pl.ds(off, L)]
                + m2_v[r % 4, pl.ds(off, L)]
            )

    def abs_load(c, i):
        row0 = base + c * CH
        return pltpu.async_copy(abs_hbm.at[pl.ds(row0, CH)], pe[i],
                                sem_abs[i])

    def x_load(c, b, i):
        row0 = base + c * CH
        return pltpu.async_copy(x_hbm.at[b, pl.ds(row0, CH)], xb[i],
                                sem_x[i])

    def out_store(c, b, i):
        row0 = base + c * CH
        return pltpu.async_copy(xb[i], out_hbm.at[b, pl.ds(row0, CH)],
                                sem_o[i])

    d_abs = [None, None]
    d_x = [None, None]
    d_o = [None, None]
    d_abs[0] = abs_load(0, 0)
    d_x[0] = x_load(0, 0, 0)

    for c in range(NCHUNK):
        A = c % 2
        d_abs[A].wait()
        if c + 1 < NCHUNK:
            d_abs[(c + 1) % 2] = abs_load(c + 1, (c + 1) % 2)

        p0 = (base + c * CH) % 12

        @plsc.parallel_loop(0, CH // GR, step=1)
        def pe_grp(g):
            js = [(p0 + g * GR + k) % 12 for k in range(GR)]

            @plsc.parallel_loop(0, D, step=L, unroll=4)
            def pe_vec(off):
                for k in range(GR):
                    plsc.addupdate(pe[A].at[g * GR + k, pl.ds(off, L)],
                                   c12_v[js[k], pl.ds(off, L)])

        for b in range(BATCH):
            gb = c * BATCH + b
            A2 = gb % 2
            d_x[A2].wait()
            if gb + 1 < NSTEP:
                B2 = (gb + 1) % 2
                if d_o[B2] is not None:
                    d_o[B2].wait()
                    d_o[B2] = None
                nc, nb = divmod(gb + 1, BATCH)
                d_x[B2] = x_load(nc, nb, B2)

            @plsc.parallel_loop(0, CH // GR, step=1)
            def add_grp(g):
                @plsc.parallel_loop(0, D, step=L, unroll=4)
                def add_vec(off):
                    for k in range(GR):
                        plsc.addupdate(xb[A2].at[g * GR + k, pl.ds(off, L)],
                                       pe[A][g * GR + k, pl.ds(off, L)])

            d_o[A2] = out_store(c, b, A2)

    for i in (0, 1):
        if d_o[i] is not None:
            d_o[i].wait()


@jax.jit
def kernel(x, abs_table, mod_table_0, mod_table_1, mod_table_2):
    mesh = plsc.VectorSubcoreMesh(core_axis_name="c", subcore_axis_name="s")
    f = pl.kernel(
        _body,
        out_type=jax.ShapeDtypeStruct((BATCH, SEQ, D), jnp.float32),
        mesh=mesh,
        scratch_types=[
            pltpu.VMEM((2, D), jnp.float32),
            pltpu.VMEM((3, D), jnp.float32),
            pltpu.VMEM((4, D), jnp.float32),
            pltpu.VMEM((12, D), jnp.float32),
            pltpu.VMEM((CH, D), jnp.float32),
            pltpu.VMEM((CH, D), jnp.float32),
            pltpu.VMEM((CH, D), jnp.float32),
            pltpu.VMEM((CH, D), jnp.float32),
            pltpu.SemaphoreType.DMA,
            pltpu.SemaphoreType.DMA,
            pltpu.SemaphoreType.DMA,
            pltpu.SemaphoreType.DMA,
            pltpu.SemaphoreType.DMA,
            pltpu.SemaphoreType.DMA,
        ],
    )
    return f(x, abs_table, mod_table_0, mod_table_1, mod_table_2)


# speedup vs baseline: 1.1661x; 1.0796x over previous
import jax
import jax.numpy as jnp
from jax import lax
from jax.experimental import pallas as pl
from jax.experimental.pallas import tpu as pltpu
from jax.experimental.pallas import tpu_sc as plsc

NC = 2
NS = 16
NW = NC * NS
L = 16

D = 768
SEQ = 8192
BATCH = 4
ROWS_PER_W = SEQ // NW
CH = 32
GR = 8
NCHUNK = ROWS_PER_W // CH
NSTEP = NCHUNK * BATCH


def _body(x_hbm, abs_hbm, m0_hbm, m1_hbm, m2_hbm, out_hbm,
          m0_v, m1_v, m2_v, c12_v, pe0_v, pe1_v, xb0_v, xb1_v,
          sem_abs0, sem_abs1, sem_x0, sem_x1, sem_o0, sem_o1):
    wid = lax.axis_index("s") * NC + lax.axis_index("c")
    base = wid * ROWS_PER_W

    pe = (pe0_v, pe1_v)
    xb = (xb0_v, xb1_v)
    sem_abs = (sem_abs0, sem_abs1)
    sem_x = (sem_x0, sem_x1)
    sem_o = (sem_o0, sem_o1)

    pltpu.sync_copy(m0_hbm, m0_v)
    pltpu.sync_copy(m1_hbm, m1_v)
    pltpu.sync_copy(m2_hbm, m2_v)
    for r in range(12):
        @plsc.parallel_loop(0, D, step=L, unroll=8)
        def c12_vec(off, r=r):
            c12_v[r, pl.ds(off, L)] = (
                m0_v[r % 2, pl.ds(off, L)]
                + m1_v[r % 3, pl.ds(off, L)]
                + m2_v[r % 4, pl.ds(off, L)]
            )

    def abs_load(c, i):
        row0 = base + c * CH
        return pltpu.async_copy(abs_hbm.at[pl.ds(row0, CH)], pe[i],
                                sem_abs[i])

    def x_load(c, b, i):
        row0 = base + c * CH
        return pltpu.async_copy(x_hbm.at[b, pl.ds(row0, CH)], xb[i],
                                sem_x[i])

    def out_store(c, b, i):
        row0 = base + c * CH
        return pltpu.async_copy(xb[i], out_hbm.at[b, pl.ds(row0, CH)],
                                sem_o[i])

    d_abs = [None, None]
    d_x = [None, None]
    d_o = [None, None]
    d_abs[0] = abs_load(0, 0)
    d_x[0] = x_load(0, 0, 0)

    for c in range(NCHUNK):
        A = c % 2
        d_abs[A].wait()
        if c + 1 < NCHUNK:
            d_abs[(c + 1) % 2] = abs_load(c + 1, (c + 1) % 2)

        p0 = (base + c * CH) % 12

        @plsc.parallel_loop(0, CH // GR, step=1)
        def pe_grp(g):
            js = [(p0 + g * GR + k) % 12 for k in range(GR)]

            @plsc.parallel_loop(0, D, step=L, unroll=2)
            def pe_vec(off):
                for k in range(GR):
                    plsc.addupdate(pe[A].at[g * GR + k, pl.ds(off, L)],
                                   c12_v[js[k], pl.ds(off, L)])

        for b in range(BATCH):
            gb = c * BATCH + b
            A2 = gb % 2
            d_x[A2].wait()
            if gb + 1 < NSTEP:
                B2 = (gb + 1) % 2
                if d_o[B2] is not None:
                    for d in d_o[B2]:
                        d.wait()
                    d_o[B2] = None
                nc, nb = divmod(gb + 1, BATCH)
                d_x[B2] = x_load(nc, nb, B2)

            row0 = base + c * CH
            half = CH // 2
            descs = []
            for hh in range(2):
                r0 = hh * half

                @plsc.parallel_loop(0, half // GR, step=1)
                def add_grp(g, r0=r0):
                    @plsc.parallel_loop(0, D, step=L, unroll=2)
                    def add_vec(off):
                        for k in range(GR):
                            plsc.addupdate(
                                xb[A2].at[r0 + g * GR + k, pl.ds(off, L)],
                                pe[A][r0 + g * GR + k, pl.ds(off, L)])

                descs.append(pltpu.async_copy(
                    xb[A2].at[pl.ds(r0, half)],
                    out_hbm.at[b, pl.ds(row0 + r0, half)], sem_o[A2]))
            d_o[A2] = descs

    for i in (0, 1):
        if d_o[i] is not None:
            for d in d_o[i]:
                d.wait()


@jax.jit
def kernel(x, abs_table, mod_table_0, mod_table_1, mod_table_2):
    mesh = plsc.VectorSubcoreMesh(core_axis_name="c", subcore_axis_name="s")
    f = pl.kernel(
        _body,
        out_type=jax.ShapeDtypeStruct((BATCH, SEQ, D), jnp.float32),
        mesh=mesh,
        scratch_types=[
            pltpu.VMEM((2, D), jnp.float32),
            pltpu.VMEM((3, D), jnp.float32),
            pltpu.VMEM((4, D), jnp.float32),
            pltpu.VMEM((12, D), jnp.float32),
            pltpu.VMEM((CH, D), jnp.float32),
            pltpu.VMEM((CH, D), jnp.float32),
            pltpu.VMEM((CH, D), jnp.float32),
            pltpu.VMEM((CH, D), jnp.float32),
            pltpu.SemaphoreType.DMA,
            pltpu.SemaphoreType.DMA,
            pltpu.SemaphoreType.DMA,
            pltpu.SemaphoreType.DMA,
            pltpu.SemaphoreType.DMA,
            pltpu.SemaphoreType.DMA,
        ],
    )
    return f(x, abs_table, mod_table_0, mod_table_1, mod_table_2)


# speedup vs baseline: 1.1672x; 1.0010x over previous
import jax
import jax.numpy as jnp
from jax import lax
from jax.experimental import pallas as pl
from jax.experimental.pallas import tpu as pltpu
from jax.experimental.pallas import tpu_sc as plsc

NC = 2
NS = 16
NW = NC * NS
L = 16

D = 768
SEQ = 8192
BATCH = 4
ROWS_PER_W = SEQ // NW
CH = 32
GR = 8
NCHUNK = ROWS_PER_W // CH
NSTEP = NCHUNK * BATCH


def _body(x_hbm, abs_hbm, m0_hbm, m1_hbm, m2_hbm, out_hbm,
          m0_v, m1_v, m2_v, c12_v, pe0_v, pe1_v, xb0_v, xb1_v,
          sem_abs0, sem_abs1, sem_x0, sem_x1, sem_o0, sem_o1):
    wid = lax.axis_index("s") * NC + lax.axis_index("c")
    base = wid * ROWS_PER_W

    pe = (pe0_v, pe1_v)
    xb = (xb0_v, xb1_v)
    sem_abs = (sem_abs0, sem_abs1)
    sem_x = (sem_x0, sem_x1)
    sem_o = (sem_o0, sem_o1)

    pltpu.sync_copy(m0_hbm, m0_v)
    pltpu.sync_copy(m1_hbm, m1_v)
    pltpu.sync_copy(m2_hbm, m2_v)
    for r in range(12):
        @plsc.parallel_loop(0, D, step=L, unroll=8)
        def c12_vec(off, r=r):
            c12_v[r, pl.ds(off, L)] = (
                m0_v[r % 2, pl.ds(off, L)]
                + m1_v[r % 3, pl.ds(off, L)]
                + m2_v[r % 4, pl.ds(off, L)]
            )

    def abs_load(c, i):
        row0 = base + c * CH
        return pltpu.async_copy(abs_hbm.at[pl.ds(row0, CH)], pe[i],
                                sem_abs[i])

    def x_load(c, b, i):
        row0 = base + c * CH
        half = CH // 2
        return [
            pltpu.async_copy(x_hbm.at[b, pl.ds(row0 + hh * half, half)],
                             xb[i].at[pl.ds(hh * half, half)], sem_x[i])
            for hh in range(2)
        ]

    def out_store(c, b, i):
        row0 = base + c * CH
        return pltpu.async_copy(xb[i], out_hbm.at[b, pl.ds(row0, CH)],
                                sem_o[i])

    d_abs = [None, None]
    d_x = [None, None]
    d_o = [None, None]
    d_abs[0] = abs_load(0, 0)
    d_x[0] = x_load(0, 0, 0)

    for c in range(NCHUNK):
        A = c % 2
        d_abs[A].wait()
        if c + 1 < NCHUNK:
            d_abs[(c + 1) % 2] = abs_load(c + 1, (c + 1) % 2)

        p0 = (base + c * CH) % 12

        @plsc.parallel_loop(0, CH // GR, step=1)
        def pe_grp(g):
            js = [(p0 + g * GR + k) % 12 for k in range(GR)]

            @plsc.parallel_loop(0, D, step=L, unroll=2)
            def pe_vec(off):
                for k in range(GR):
                    plsc.addupdate(pe[A].at[g * GR + k, pl.ds(off, L)],
                                   c12_v[js[k], pl.ds(off, L)])

        for b in range(BATCH):
            gb = c * BATCH + b
            A2 = gb % 2
            if gb + 1 < NSTEP:
                B2 = (gb + 1) % 2
                if d_o[B2] is not None:
                    for d in d_o[B2]:
                        d.wait()
                    d_o[B2] = None
                nc, nb = divmod(gb + 1, BATCH)
                d_x[B2] = x_load(nc, nb, B2)

            row0 = base + c * CH
            half = CH // 2
            descs = []
            for hh in range(2):
                r0 = hh * half
                d_x[A2][hh].wait()

                @plsc.parallel_loop(0, half // GR, step=1)
                def add_grp(g, r0=r0):
                    @plsc.parallel_loop(0, D, step=L, unroll=2)
                    def add_vec(off):
                        for k in range(GR):
                            plsc.addupdate(
                                xb[A2].at[r0 + g * GR + k, pl.ds(off, L)],
                                pe[A][r0 + g * GR + k, pl.ds(off, L)])

                descs.append(pltpu.async_copy(
                    xb[A2].at[pl.ds(r0, half)],
                    out_hbm.at[b, pl.ds(row0 + r0, half)], sem_o[A2]))
            d_o[A2] = descs

    for i in (0, 1):
        if d_o[i] is not None:
            for d in d_o[i]:
                d.wait()


@jax.jit
def kernel(x, abs_table, mod_table_0, mod_table_1, mod_table_2):
    mesh = plsc.VectorSubcoreMesh(core_axis_name="c", subcore_axis_name="s")
    f = pl.kernel(
        _body,
        out_type=jax.ShapeDtypeStruct((BATCH, SEQ, D), jnp.float32),
        mesh=mesh,
        scratch_types=[
            pltpu.VMEM((2, D), jnp.float32),
            pltpu.VMEM((3, D), jnp.float32),
            pltpu.VMEM((4, D), jnp.float32),
            pltpu.VMEM((12, D), jnp.float32),
            pltpu.VMEM((CH, D), jnp.float32),
            pltpu.VMEM((CH, D), jnp.float32),
            pltpu.VMEM((CH, D), jnp.float32),
            pltpu.VMEM((CH, D), jnp.float32),
            pltpu.SemaphoreType.DMA,
            pltpu.SemaphoreType.DMA,
            pltpu.SemaphoreType.DMA,
            pltpu.SemaphoreType.DMA,
            pltpu.SemaphoreType.DMA,
            pltpu.SemaphoreType.DMA,
        ],
    )
    return f(x, abs_table, mod_table_0, mod_table_1, mod_table_2)


# speedup vs baseline: 1.1678x; 1.0005x over previous
import jax
import jax.numpy as jnp
from jax import lax
from jax.experimental import pallas as pl
from jax.experimental.pallas import tpu as pltpu
from jax.experimental.pallas import tpu_sc as plsc

NC = 2
NS = 16
NW = NC * NS
L = 16

D = 768
SEQ = 8192
BATCH = 4
ROWS_PER_W = SEQ // NW
CH = 32
GR = 8
NCHUNK = ROWS_PER_W // CH
NSTEP = NCHUNK * BATCH


def _body(x_hbm, abs_hbm, m0_hbm, m1_hbm, m2_hbm, out_hbm,
          m0_v, m1_v, m2_v, c12_v, pe0_v, pe1_v, xb0_v, xb1_v,
          sem_abs0, sem_abs1, sem_x0, sem_x1, sem_o0, sem_o1):
    wid = lax.axis_index("s") * NC + lax.axis_index("c")
    base = wid * ROWS_PER_W

    pe = (pe0_v, pe1_v)
    xb = (xb0_v, xb1_v)
    sem_abs = (sem_abs0, sem_abs1)
    sem_x = (sem_x0, sem_x1)
    sem_o = (sem_o0, sem_o1)

    pltpu.sync_copy(m0_hbm, m0_v)
    pltpu.sync_copy(m1_hbm, m1_v)
    pltpu.sync_copy(m2_hbm, m2_v)
    for r in range(12):
        @plsc.parallel_loop(0, D, step=L, unroll=8)
        def c12_vec(off, r=r):
            c12_v[r, pl.ds(off, L)] = (
                m0_v[r % 2, pl.ds(off, L)]
                + m1_v[r % 3, pl.ds(off, L)]
                + m2_v[r % 4, pl.ds(off, L)]
            )

    def abs_load(c, i):
        row0 = base + c * CH
        return pltpu.async_copy(abs_hbm.at[pl.ds(row0, CH)], pe[i],
                                sem_abs[i])

    def x_load(c, b, i):
        row0 = base + c * CH
        half = CH // 2
        return [
            pltpu.async_copy(x_hbm.at[b, pl.ds(row0 + hh * half, half)],
                             xb[i].at[pl.ds(hh * half, half)], sem_x[i])
            for hh in range(2)
        ]

    d_abs = [None, None]
    d_x = [None, None]
    d_o = [None, None]
    d_abs[0] = abs_load(0, 0)
    d_x[0] = x_load(0, 0, 0)

    for c in range(NCHUNK):
        A = c % 2
        d_abs[A].wait()
        if c + 1 < NCHUNK:
            d_abs[(c + 1) % 2] = abs_load(c + 1, (c + 1) % 2)

        p0 = (base + c * CH) % 12

        @plsc.parallel_loop(0, CH // GR, step=1)
        def pe_grp(g):
            js = [(p0 + g * GR + k) % 12 for k in range(GR)]

            @plsc.parallel_loop(0, D, step=L, unroll=2)
            def pe_vec(off):
                for k in range(GR):
                    plsc.addupdate(pe[A].at[g * GR + k, pl.ds(off, L)],
                                   c12_v[js[k], pl.ds(off, L)])

        for b in range(BATCH):
            gb = c * BATCH + b
            A2 = gb % 2
            if gb + 1 < NSTEP:
                B2 = (gb + 1) % 2
                if d_o[B2] is not None:
                    for d in d_o[B2]:
                        d.wait()
                    d_o[B2] = None
                nc, nb = divmod(gb + 1, BATCH)
                d_x[B2] = x_load(nc, nb, B2)

            row0 = base + c * CH
            half = CH // 2
            descs = []
            for hh in range(2):
                r0 = hh * half
                d_x[A2][hh].wait()

                @plsc.parallel_loop(0, half // GR, step=1)
                def add_grp(g, r0=r0):
                    @plsc.parallel_loop(0, D, step=L, unroll=2)
                    def add_vec(off):
                        for k in range(GR):
                            plsc.addupdate(
                                xb[A2].at[r0 + g * GR + k, pl.ds(off, L)],
                                pe[A][r0 + g * GR + k, pl.ds(off, L)])

                descs.append(pltpu.async_copy(
                    xb[A2].at[pl.ds(r0, half)],
                    out_hbm.at[b, pl.ds(row0 + r0, half)], sem_o[A2]))
            d_o[A2] = descs

    for i in (0, 1):
        if d_o[i] is not None:
            for d in d_o[i]:
                d.wait()


@jax.jit
def kernel(x, abs_table, mod_table_0, mod_table_1, mod_table_2):
    mesh = plsc.VectorSubcoreMesh(core_axis_name="c", subcore_axis_name="s")
    f = pl.kernel(
        _body,
        out_type=jax.ShapeDtypeStruct((BATCH, SEQ, D), jnp.float32),
        mesh=mesh,
        scratch_types=[
            pltpu.VMEM((2, D), jnp.float32),
            pltpu.VMEM((3, D), jnp.float32),
            pltpu.VMEM((4, D), jnp.float32),
            pltpu.VMEM((12, D), jnp.float32),
            pltpu.VMEM((CH, D), jnp.float32),
            pltpu.VMEM((CH, D), jnp.float32),
            pltpu.VMEM((CH, D), jnp.float32),
            pltpu.VMEM((CH, D), jnp.float32),
            pltpu.SemaphoreType.DMA,
            pltpu.SemaphoreType.DMA,
            pltpu.SemaphoreType.DMA,
            pltpu.SemaphoreType.DMA,
            pltpu.SemaphoreType.DMA,
            pltpu.SemaphoreType.DMA,
        ],
    )
    return f(x, abs_table, mod_table_0, mod_table_1, mod_table_2)
